# Initial kernel scaffold; baseline (speedup 1.0000x reference)
import jax
import jax.numpy as jnp
from jax.experimental import pallas as pl


def kernel(x, edge_index, batch, params):
    raise NotImplementedError("write your pallas kernel here")


# speedup vs baseline: 20.9070x; 20.9070x over previous
import functools
import math

import jax
import jax.numpy as jnp
from jax import lax
from jax.experimental import pallas as pl
from jax.experimental.pallas import tpu as pltpu
from jax.experimental.pallas import tpu_sc as plsc

N0 = 10000
NP = 10240
NB = NP // 128
NACC = 10368
E0 = 320000
NC, NS = 2, 16
NW = NC * NS
W = 128
NWIN = 79
EPAD = NW * NWIN * W
DEPTH = 4
EPS = 1e-5
INT_MIN = -2147483648


def _layer_sizes():
    ns = [N0]
    for _ in range(DEPTH):
        ns.append(int(math.ceil(0.8 * ns[-1])))
    return ns


_NS_LIST = _layer_sizes()


def _mm_body(x_ref, w_ref, b_ref, o_ref):
    o_ref[...] = jnp.dot(x_ref[...], w_ref[...],
                         preferred_element_type=jnp.float32) + b_ref[...]


def _tc_mm(x, w, b):
    return pl.pallas_call(
        _mm_body, out_shape=jax.ShapeDtypeStruct((NP, 128), jnp.float32),
    )(x, w, b)


def _bnmm_body(x_ref, mean_ref, var_ref, g_ref, be_ref, w_ref, b_ref, o_ref):
    y = g_ref[...] * (x_ref[...] - mean_ref[...]) \
        / jnp.sqrt(var_ref[...] + EPS) + be_ref[...]
    y = jnp.maximum(y, 0.0)
    o_ref[...] = jnp.dot(y, w_ref[...],
                         preferred_element_type=jnp.float32) + b_ref[...]


def _tc_bnmm(x, mean, var, g, be, w, b):
    return pl.pallas_call(
        _bnmm_body, out_shape=jax.ShapeDtypeStruct((NP, 128), jnp.float32),
    )(x, mean.reshape(1, 128), var.reshape(1, 128), g, be, w, b)


def _bn_stats(u, m_col, i, k_cnt):
    if i == 0:
        return jnp.mean(u[:N0], axis=0), jnp.var(u[:N0], axis=0)
    s1 = jnp.sum(u * m_col, axis=0) / k_cnt
    d = u - s1
    s2 = jnp.sum(d * d * m_col, axis=0) / k_cnt
    return s1, s2


def _scale_body(deg_ref, hc_ref, hp_ref):
    deg = deg_ref[0] + deg_ref[1]
    dinv = 1.0 / jnp.sqrt(deg[:NB] + 2.0)
    hc3 = hc_ref[...].reshape(NB, 128, 128)
    hp_ref[...] = (hc3 * dinv[:, :, None]).reshape(NP, 128)


def _tc_scale(deg3, hc):
    return pl.pallas_call(
        _scale_body,
        out_shape=jax.ShapeDtypeStruct((NP, 128), jnp.float32),
    )(deg3, hc)


def _post_body(k_next, agg_ref, deg_ref, hc_ref, mcol_ref, cb_ref, pw_ref,
               x2_ref, mnext_ref, r_ref):
    deg = deg_ref[0] + deg_ref[1]
    dinv = 1.0 / jnp.sqrt(deg[:NB] + 2.0)
    dinv3 = dinv[:, :, None]
    a3 = (agg_ref[0, :NP, :] + agg_ref[1, :NP, :]).reshape(NB, 128, 128)
    hc3 = hc_ref[...].reshape(NB, 128, 128)
    gcn = (dinv3 * a3 + (2.0 * dinv3 * dinv3) * hc3).reshape(NP, 128)
    gcn = gcn + cb_ref[...]

    pw = pw_ref[...]
    pn = jnp.sqrt(jnp.sum(pw * pw))
    s = jnp.tanh(jnp.dot(gcn, pw, preferred_element_type=jnp.float32) / pn)

    mcol = mcol_ref[...]
    imin = jnp.int32(INT_MIN)
    b = lax.bitcast_convert_type(s, jnp.int32)
    key = jnp.where(b >= 0, b, imin - b)
    key = jnp.where(mcol > 0.5, key, imin)

    k_i32 = jnp.int32(k_next)

    def cnt_gt(t):
        return jnp.sum((key > t).astype(jnp.int32))

    def bis1(_, c):
        lo, hi = c
        mid = (lo >> 1) + (hi >> 1) + (lo & hi & 1)
        p = cnt_gt(mid) < k_i32
        return (jnp.where(p, lo, mid + 1), jnp.where(p, mid, hi))

    lo, hi = lax.fori_loop(0, 33, bis1, (imin, jnp.int32(2147483647)))
    t = lo
    g = cnt_gt(t)
    need = k_i32 - g
    eq = key == t
    idx = lax.broadcasted_iota(jnp.int32, (NP, 1), 0)

    def bis2(_, c):
        lo2, hi2 = c
        mid = (lo2 + hi2) >> 1
        q = jnp.sum((eq & (idx < mid)).astype(jnp.int32)) >= need
        return (jnp.where(q, lo2, mid + 1), jnp.where(q, mid, hi2))

    lo2, _hi2 = lax.fori_loop(0, 15, bis2, (jnp.int32(0), jnp.int32(NP)))
    mnext = ((key > t) | (eq & (idx < lo2))).astype(jnp.float32)

    x2 = gcn * s * mnext
    x2_ref[...] = x2
    mnext_ref[...] = mnext
    mx = jnp.max(jnp.where(mnext > 0.5, x2, -jnp.inf), axis=0, keepdims=True)
    mn = jnp.sum(x2, axis=0, keepdims=True) * (1.0 / k_next)
    r_ref[...] = jnp.concatenate([mx, mn], axis=1)


def _tc_post(agg, deg3, hc, mcol, cb, pw, k_next):
    return pl.pallas_call(
        functools.partial(_post_body, float(k_next)),
        out_shape=[
            jax.ShapeDtypeStruct((NP, 128), jnp.float32),
            jax.ShapeDtypeStruct((NP, 1), jnp.float32),
            jax.ShapeDtypeStruct((1, 256), jnp.float32),
        ],
        compiler_params=pltpu.CompilerParams(
            vmem_limit_bytes=100 * 1024 * 1024),
    )(agg, deg3, hc, mcol, cb, pw)


def _head_body(r0_ref, r1_ref, r2_ref, r3_ref, w1_ref, b1_ref, w2_ref, b2_ref,
               w3_ref, b3_ref, z_ref):
    r = (r0_ref[...] + r1_ref[...] + r2_ref[...] + r3_ref[...]) * 0.25
    z = jnp.maximum(jnp.dot(r, w1_ref[...], preferred_element_type=jnp.float32)
                    + b1_ref[...], 0.0)
    z = jnp.maximum(jnp.dot(z, w2_ref[...], preferred_element_type=jnp.float32)
                    + b2_ref[...], 0.0)
    z_ref[...] = jnp.dot(z, w3_ref[...], preferred_element_type=jnp.float32) \
        + b3_ref[...]


def _tc_head(rs, params):
    return pl.pallas_call(
        _head_body,
        out_shape=jax.ShapeDtypeStruct((1, 16), jnp.float32),
    )(rs[0], rs[1], rs[2], rs[3],
      params['pred_w1'], params['pred_b1'],
      params['pred_w2'], params['pred_b2'],
      params['pred_w3'], params['pred_b3'])


@functools.cache
def _sc_prep_kernel():
    return functools.partial(
        pl.kernel,
        out_type=[
            jax.ShapeDtypeStruct((NC, NACC), jnp.float32),
            jax.ShapeDtypeStruct((NW, NWIN, W), jnp.int32),
        ],
        mesh=plsc.VectorSubcoreMesh(core_axis_name="c", subcore_axis_name="s",
                                    num_cores=NC, num_subcores=NS),
        scratch_types=[
            pltpu.VMEM((NP,), jnp.float32),
            pltpu.VMEM((W,), jnp.int32),
            pltpu.VMEM((W,), jnp.int32),
            pltpu.VMEM((W,), jnp.int32),
            pltpu.VMEM((W,), jnp.float32),
            pltpu.VMEM((NACC,), jnp.float32),
            pltpu.VMEM_SHARED((NACC,), jnp.float32),
        ],
        compiler_params=pltpu.CompilerParams(needs_layout_passes=False),
    )(_sc_prep_body)


def _sc_prep(srcR, dstR, m_flat):
    return _sc_prep_kernel()(srcR, dstR, m_flat)


def _sc_prep_body(src_hbm, dst_hbm, m_hbm, deg_hbm, deff_hbm,
                  m_v, s_v, d_v, de_v, va_v, z_v, deg_sh):
    cid = lax.axis_index("c")
    sid = lax.axis_index("s")
    wid = sid * NC + cid

    def zbody(i, _):
        z_v[pl.ds(pl.multiple_of(i * 16, 16), 16)] = jnp.zeros((16,), jnp.float32)
        return 0
    lax.fori_loop(0, NACC // 16, zbody, 0)

    @pl.when(sid == 0)
    def _():
        pltpu.sync_copy(z_v, deg_sh)

    pltpu.sync_copy(m_hbm, m_v)
    plsc.subcore_barrier()

    def wbody(w, _):
        pltpu.sync_copy(src_hbm.at[wid, w], s_v)
        pltpu.sync_copy(dst_hbm.at[wid, w], d_v)
        for g in range(W // 16):
            sl = pl.ds(g * 16, 16)
            s16 = s_v[sl]
            d16 = d_v[sl]
            ms = plsc.load_gather(m_v, [s16])
            md = plsc.load_gather(m_v, [d16])
            valid = ms * md
            dummy = jnp.int32(NP) + lax.iota(jnp.int32, 16)
            de_v[sl] = jnp.where(valid > 0.5, d16, dummy)
            va_v[sl] = valid
        pltpu.sync_copy(de_v, deff_hbm.at[wid, w])
        pltpu.sync_copy(va_v, deg_sh.at[de_v], add=True)
        return 0

    lax.fori_loop(0, NWIN, wbody, 0)
    plsc.subcore_barrier()

    @pl.when(sid == 0)
    def _():
        pltpu.sync_copy(deg_sh, deg_hbm.at[cid])


_RPS = NACC // NS


@functools.cache
def _sc_msg_kernel():
    return functools.partial(
        pl.kernel,
        out_type=jax.ShapeDtypeStruct((NC, NACC, 128), jnp.float32),
        mesh=plsc.VectorSubcoreMesh(core_axis_name="c", subcore_axis_name="s",
                                    num_cores=NC, num_subcores=NS),
        scratch_types=[
            pltpu.VMEM((W,), jnp.int32),
            pltpu.VMEM((W,), jnp.int32),
            pltpu.VMEM((W, 128), jnp.float32),
            pltpu.VMEM((8, 128), jnp.float32),
            pltpu.VMEM_SHARED((NACC, 128), jnp.float32),
            pltpu.SemaphoreType.DMA,
        ],
        compiler_params=pltpu.CompilerParams(needs_layout_passes=False),
    )(_sc_msg_body)


def _sc_msg(srcR, deff, hp):
    return _sc_msg_kernel()(srcR, deff, hp)


def _sc_msg_body(src_hbm, deff_hbm, hp_hbm, agg_hbm,
                 s_v, d_v, rows_v, zr_v, acc_sh, sem):
    cid = lax.axis_index("c")
    sid = lax.axis_index("s")
    wid = sid * NC + cid

    for r in range(8):
        for g in range(8):
            zr_v[r, pl.ds(g * 16, 16)] = jnp.zeros((16,), jnp.float32)

    base = sid * _RPS

    def zbody(j, _):
        acc_sh_rows = acc_sh.at[pl.ds(pl.multiple_of(base + j * 8, 8), 8)]
        pltpu.sync_copy(zr_v, acc_sh_rows)
        return 0
    lax.fori_loop(0, _RPS // 8, zbody, 0)
    plsc.subcore_barrier()

    def wbody(w, _):
        pltpu.sync_copy(src_hbm.at[wid, w], s_v)
        pltpu.sync_copy(deff_hbm.at[wid, w], d_v)
        pltpu.async_copy(hp_hbm.at[s_v], rows_v, sem).wait()
        pltpu.sync_copy(rows_v, acc_sh.at[d_v], add=True)
        return 0

    lax.fori_loop(0, NWIN, wbody, 0)
    plsc.subcore_barrier()
    pltpu.sync_copy(acc_sh.at[pl.ds(base, _RPS)],
                    agg_hbm.at[cid, pl.ds(base, _RPS)])


def kernel(x, edge_index, batch, params):
    del batch
    x_p = jnp.pad(x.astype(jnp.float32), ((0, NP - N0), (0, 0)))

    src = edge_index[0].astype(jnp.int32)
    dst = edge_index[1].astype(jnp.int32)
    pad_idx = (NP - 16) + (jnp.arange(EPAD - E0, dtype=jnp.int32) % 16)
    srcR = jnp.concatenate([src, pad_idx]).reshape(NW, NWIN, W)
    dstR = jnp.concatenate([dst, pad_idx]).reshape(NW, NWIN, W)

    m_col = (jnp.arange(NP, dtype=jnp.int32) < N0).astype(jnp.float32)[:, None]

    h = x_p
    readouts = []
    for i in range(DEPTH):
        p = params['layer%d' % i]
        n_i, k_next = _NS_LIST[i], _NS_LIST[i + 1]
        u1 = _tc_mm(h, p['t_w1'], p['t_b1'])
        m1, v1 = _bn_stats(u1, m_col, i, n_i)
        u2 = _tc_bnmm(u1, m1, v1, p['t_g1'], p['t_be1'], p['t_w2'], p['t_b2'])
        m2, v2 = _bn_stats(u2, m_col, i, n_i)
        hc = _tc_bnmm(u2, m2, v2, p['t_g2'], p['t_be2'], p['c_w'],
                      jnp.zeros((128,), jnp.float32))
        deg, deff = _sc_prep(srcR, dstR, m_col.reshape(NP))
        deg3 = deg.reshape(NC, NACC // 128, 128)
        hp = _tc_scale(deg3, hc)
        agg = _sc_msg(srcR, deff, hp)
        h, m_col, r = _tc_post(agg, deg3, hc, m_col, p['c_b'],
                               p['p_w'].reshape(128, 1), k_next)
        readouts.append(r)

    return _tc_head(readouts, params)
